# baseline (speedup 1.0000x reference)
import functools

import jax
import jax.numpy as jnp
from jax import lax
from jax.experimental import pallas as pl
from jax.experimental.pallas import tpu as pltpu
from jax.experimental.pallas import tpu_sc as plsc

SEQ = 50
DIM = 64
NB = 16384
NUM_WORKERS = 32
BBLK = NB // NUM_WORKERS
HALF = 256
N_CHUNKS = SEQ * 2


def _transpose_chunk(rows, tbuf, pvecs, rvecs):
    for cl in range(2):
        clv = jnp.full((16,), cl, jnp.int32)

        def body(bl, carry):
            b = cl * 128 + bl
            for c0 in range(4):
                vals = rows[b, pl.ds(c0 * 16, 16)]
                qv = pvecs[c0] + bl
                plsc.store_scatter(tbuf, [rvecs[c0], clv, qv], vals)
            return carry

        lax.fori_loop(0, 128, body, 0, unroll=4)


def _body(table_hbm, idst_hbm, out_hbm,
          idx_all, rows0, rows1, tb0, tb1,
          gsem0, gsem1, osem0, osem1):
    wid = lax.axis_index("s") * 2 + lax.axis_index("c")
    b0 = wid * BBLK
    c0tile = wid * 4

    pltpu.sync_copy(idst_hbm.at[:, pl.ds(b0, BBLK)], idx_all)

    iota = lax.iota(jnp.int32, 16)
    pvecs = [((iota + c0 * 16) % 8) * 128 for c0 in range(4)]
    rvecs = [(iota + c0 * 16) // 8 for c0 in range(4)]

    rows = (rows0, rows1)
    tbs = (tb0, tb1)
    gsem = (gsem0, gsem1)
    osem = (osem0, osem1)

    def idx_ref(chunk_i):
        s = chunk_i // 2
        h = chunk_i % 2
        return idx_all.at[s, pl.ds(h * HALF, HALF)]

    def start_gather(chunk_i, p):
        pltpu.async_copy(table_hbm.at[idx_ref(chunk_i)], rows[p], gsem[p])

    def wait_gather(chunk_i, p):
        pltpu.make_async_copy(table_hbm.at[idx_ref(chunk_i)], rows[p],
                              gsem[p]).wait()

    def start_write(chunk_i, p):
        s = chunk_i // 2
        h = chunk_i % 2
        for r in range(8):
            pltpu.async_copy(tbs[p].at[r],
                             out_hbm.at[s, r, pl.ds(c0tile + h * 2, 2)],
                             osem[p])

    def wait_write(chunk_i, p):
        s = chunk_i // 2
        h = chunk_i % 2
        for r in range(8):
            pltpu.make_async_copy(tbs[p].at[r],
                                  out_hbm.at[s, r, pl.ds(c0tile + h * 2, 2)],
                                  osem[p]).wait()

    def step(gc, p, first):
        wait_gather(gc, p)
        if not first:
            wait_write(gc - 2, p)
        _transpose_chunk(rows[p], tbs[p], pvecs, rvecs)
        nxt = jnp.minimum(gc + 2, N_CHUNKS - 1)
        start_gather(nxt, p)
        start_write(gc, p)

    start_gather(0, 0)
    start_gather(1, 1)
    step(0, 0, True)
    step(1, 1, True)

    def pair(k, carry):
        g = 2 + 2 * k
        for b in range(2):
            step(g + b, b, False)
        return carry

    lax.fori_loop(0, (N_CHUNKS - 2) // 2, pair, 0)

    wait_gather(N_CHUNKS - 1, 0)
    wait_gather(N_CHUNKS - 1, 1)
    wait_write(N_CHUNKS - 2, 0)
    wait_write(N_CHUNKS - 1, 1)


def kernel(ids, table):
    ids_t = ids.T
    mesh = plsc.VectorSubcoreMesh(core_axis_name="c", subcore_axis_name="s")
    run = pl.kernel(
        _body,
        out_type=jax.ShapeDtypeStruct((SEQ, 8, 128, 1024), jnp.float32),
        mesh=mesh,
        scratch_types=[
            pltpu.VMEM((SEQ, BBLK), jnp.int32),
            pltpu.VMEM((HALF, DIM), jnp.float32),
            pltpu.VMEM((HALF, DIM), jnp.float32),
            pltpu.VMEM((8, 2, 1024), jnp.float32),
            pltpu.VMEM((8, 2, 1024), jnp.float32),
            pltpu.SemaphoreType.DMA,
            pltpu.SemaphoreType.DMA,
            pltpu.SemaphoreType.DMA,
            pltpu.SemaphoreType.DMA,
        ],
        compiler_params=pltpu.CompilerParams(use_tc_tiling_on_sc=False, needs_layout_passes=False),
    )
    out5 = run(table, ids_t)
    out_t = (out5.reshape(SEQ, 8, 128, 8, 128)
             .transpose(0, 1, 3, 2, 4)
             .reshape(SEQ, DIM, NB))
    return jnp.transpose(out_t, (2, 0, 1))


# speedup vs baseline: 1.0069x; 1.0069x over previous
import functools

import jax
import jax.numpy as jnp
from jax import lax
from jax.experimental import pallas as pl
from jax.experimental.pallas import tpu as pltpu
from jax.experimental.pallas import tpu_sc as plsc

SEQ = 50
DIM = 64
NB = 16384
NUM_WORKERS = 32
BBLK = NB // NUM_WORKERS
HALF = 256
N_CHUNKS = SEQ * 2


def _transpose_chunk(rows, tbuf, rvecs, clvecs, qbases):
    for cl in range(2):
        def body(bl, carry):
            b = cl * 128 + bl
            for c0 in range(4):
                vals = rows[b, pl.ds(c0 * 16, 16)]
                plsc.store_scatter(
                    tbuf, [rvecs[c0], clvecs[cl], qbases[cl][c0] + bl], vals)
            return carry
        lax.fori_loop(0, 128, body, 0, unroll=8)


def _body(table_hbm, idst_hbm, out_hbm,
          idx_all, rows0, rows1, tb0, tb1,
          gsem0, gsem1, osem0, osem1):
    wid = lax.axis_index("s") * 2 + lax.axis_index("c")
    b0 = wid * BBLK
    c0tile = wid * 4

    pltpu.sync_copy(idst_hbm.at[:, pl.ds(b0, BBLK)], idx_all)

    iota = lax.iota(jnp.int32, 16)
    rvecs = [(iota + c0 * 16) // 8 for c0 in range(4)]
    qbases = [[((iota + c0 * 16) % 8) * 128 for c0 in range(4)]
              for cl in range(2)]
    clvecs = [jnp.full((16,), cl, jnp.int32) for cl in range(2)]

    rows = (rows0, rows1)
    tbs = (tb0, tb1)
    gsem = (gsem0, gsem1)
    osem = (osem0, osem1)

    def idx_ref(chunk_i):
        s = chunk_i // 2
        h = chunk_i % 2
        return idx_all.at[s, pl.ds(h * HALF, HALF)]

    def start_gather(chunk_i, p):
        pltpu.async_copy(table_hbm.at[idx_ref(chunk_i)], rows[p], gsem[p])

    def wait_gather(chunk_i, p):
        pltpu.make_async_copy(table_hbm.at[idx_ref(chunk_i)], rows[p],
                              gsem[p]).wait()

    def out_ref(chunk_i):
        s = chunk_i // 2
        h = chunk_i % 2
        return out_hbm.at[s, :, pl.ds(c0tile + h * 2, 2)]

    def start_write(chunk_i, p):
        pltpu.async_copy(tbs[p], out_ref(chunk_i), osem[p])

    def wait_write(chunk_i, p):
        pltpu.make_async_copy(tbs[p], out_ref(chunk_i), osem[p]).wait()

    def step(gc, p, first):
        wait_gather(gc, p)
        if not first:
            wait_write(gc - 2, p)
        _transpose_chunk(rows[p], tbs[p], rvecs, clvecs, qbases)
        nxt = jnp.minimum(gc + 2, N_CHUNKS - 1)
        start_gather(nxt, p)
        start_write(gc, p)

    start_gather(0, 0)
    start_gather(1, 1)
    step(0, 0, True)
    step(1, 1, True)

    def pair(k, carry):
        g = 2 + 2 * k
        for b in range(2):
            step(g + b, b, False)
        return carry

    lax.fori_loop(0, (N_CHUNKS - 2) // 2, pair, 0)

    wait_gather(N_CHUNKS - 1, 0)
    wait_gather(N_CHUNKS - 1, 1)
    wait_write(N_CHUNKS - 2, 0)
    wait_write(N_CHUNKS - 1, 1)


def kernel(ids, table):
    ids_t = ids.T
    mesh = plsc.VectorSubcoreMesh(core_axis_name="c", subcore_axis_name="s")
    run = pl.kernel(
        _body,
        out_type=jax.ShapeDtypeStruct((SEQ, 8, 128, 1024), jnp.float32),
        mesh=mesh,
        scratch_types=[
            pltpu.VMEM((SEQ, BBLK), jnp.int32),
            pltpu.VMEM((HALF, DIM), jnp.float32),
            pltpu.VMEM((HALF, DIM), jnp.float32),
            pltpu.VMEM((8, 2, 1024), jnp.float32),
            pltpu.VMEM((8, 2, 1024), jnp.float32),
            pltpu.SemaphoreType.DMA,
            pltpu.SemaphoreType.DMA,
            pltpu.SemaphoreType.DMA,
            pltpu.SemaphoreType.DMA,
        ],
        compiler_params=pltpu.CompilerParams(use_tc_tiling_on_sc=False,
                                             needs_layout_passes=False),
    )
    out5 = run(table, ids_t)
    out_t = (out5.reshape(SEQ, 8, 128, 8, 128)
             .transpose(0, 1, 3, 2, 4)
             .reshape(SEQ, DIM, NB))
    return jnp.transpose(out_t, (2, 0, 1))


# speedup vs baseline: 1.2146x; 1.2062x over previous
import functools

import jax
import jax.numpy as jnp
from jax import lax
from jax.experimental import pallas as pl
from jax.experimental.pallas import tpu as pltpu
from jax.experimental.pallas import tpu_sc as plsc

SEQ = 50
DIM = 64
NB = 16384
NUM_WORKERS = 32
BBLK = NB // NUM_WORKERS
HALF = 256
N_CHUNKS = SEQ * 2


def _transpose_chunk(rows, tbuf, rvecs, clvecs, qbases):
    for cl in range(2):
        @plsc.parallel_loop(0, 128, unroll=8)
        def _(bl):
            b = cl * 128 + bl
            for c0 in range(4):
                vals = rows[b, pl.ds(c0 * 16, 16)]
                plsc.store_scatter(
                    tbuf, [rvecs[c0], clvecs[cl], qbases[cl][c0] + bl], vals)


def _body(table_hbm, idst_hbm, out_hbm,
          idx_all, rows0, rows1, tb0, tb1,
          gsem0, gsem1, osem0, osem1):
    wid = lax.axis_index("s") * 2 + lax.axis_index("c")
    b0 = wid * BBLK
    c0tile = wid * 4

    pltpu.sync_copy(idst_hbm.at[:, pl.ds(b0, BBLK)], idx_all)

    iota = lax.iota(jnp.int32, 16)
    rvecs = [(iota + c0 * 16) // 8 for c0 in range(4)]
    qbases = [[((iota + c0 * 16) % 8) * 128 for c0 in range(4)]
              for cl in range(2)]
    clvecs = [jnp.full((16,), cl, jnp.int32) for cl in range(2)]

    rows = (rows0, rows1)
    tbs = (tb0, tb1)
    gsem = (gsem0, gsem1)
    osem = (osem0, osem1)

    def idx_ref(chunk_i):
        s = chunk_i // 2
        h = chunk_i % 2
        return idx_all.at[s, pl.ds(h * HALF, HALF)]

    def start_gather(chunk_i, p):
        pltpu.async_copy(table_hbm.at[idx_ref(chunk_i)], rows[p], gsem[p])

    def wait_gather(chunk_i, p):
        pltpu.make_async_copy(table_hbm.at[idx_ref(chunk_i)], rows[p],
                              gsem[p]).wait()

    def out_ref(chunk_i):
        s = chunk_i // 2
        h = chunk_i % 2
        return out_hbm.at[s, :, pl.ds(c0tile + h * 2, 2)]

    def start_write(chunk_i, p):
        pltpu.async_copy(tbs[p], out_ref(chunk_i), osem[p])

    def wait_write(chunk_i, p):
        pltpu.make_async_copy(tbs[p], out_ref(chunk_i), osem[p]).wait()

    def step(gc, p, first):
        wait_gather(gc, p)
        if not first:
            wait_write(gc - 2, p)
        _transpose_chunk(rows[p], tbs[p], rvecs, clvecs, qbases)
        nxt = jnp.minimum(gc + 2, N_CHUNKS - 1)
        start_gather(nxt, p)
        start_write(gc, p)

    start_gather(0, 0)
    start_gather(1, 1)
    step(0, 0, True)
    step(1, 1, True)

    def pair(k, carry):
        g = 2 + 2 * k
        for b in range(2):
            step(g + b, b, False)
        return carry

    lax.fori_loop(0, (N_CHUNKS - 2) // 2, pair, 0)

    wait_gather(N_CHUNKS - 1, 0)
    wait_gather(N_CHUNKS - 1, 1)
    wait_write(N_CHUNKS - 2, 0)
    wait_write(N_CHUNKS - 1, 1)


def kernel(ids, table):
    ids_t = ids.T
    mesh = plsc.VectorSubcoreMesh(core_axis_name="c", subcore_axis_name="s")
    run = pl.kernel(
        _body,
        out_type=jax.ShapeDtypeStruct((SEQ, 8, 128, 1024), jnp.float32),
        mesh=mesh,
        scratch_types=[
            pltpu.VMEM((SEQ, BBLK), jnp.int32),
            pltpu.VMEM((HALF, DIM), jnp.float32),
            pltpu.VMEM((HALF, DIM), jnp.float32),
            pltpu.VMEM((8, 2, 1024), jnp.float32),
            pltpu.VMEM((8, 2, 1024), jnp.float32),
            pltpu.SemaphoreType.DMA,
            pltpu.SemaphoreType.DMA,
            pltpu.SemaphoreType.DMA,
            pltpu.SemaphoreType.DMA,
        ],
        compiler_params=pltpu.CompilerParams(use_tc_tiling_on_sc=False,
                                             needs_layout_passes=False),
    )
    out5 = run(table, ids_t)
    out_t = (out5.reshape(SEQ, 8, 128, 8, 128)
             .transpose(0, 1, 3, 2, 4)
             .reshape(SEQ, DIM, NB))
    return jnp.transpose(out_t, (2, 0, 1))


# speedup vs baseline: 1.2760x; 1.0506x over previous
import functools

import jax
import jax.numpy as jnp
from jax import lax
from jax.experimental import pallas as pl
from jax.experimental.pallas import tpu as pltpu
from jax.experimental.pallas import tpu_sc as plsc

SEQ = 50
DIM = 64
NB = 16384
NUM_WORKERS = 32
BBLK = NB // NUM_WORKERS
HALF = 256
N_CHUNKS = SEQ * 2


def _transpose_chunk(rows, tbuf, bvecs):
    for cl in range(2):
        @plsc.parallel_loop(0, 64, unroll=4)
        def _(cr):
            rr = lax.rem(cr, 8) * 128
            rq = lax.div(cr, 8)
            cv = jnp.full((16,), 0, jnp.int32) + cr
            for bl0 in range(8):
                v = plsc.load_gather(rows, [bvecs[cl][bl0], cv])
                tbuf[rq, cl, pl.ds(rr + bl0 * 16, 16)] = v


def _body(table_hbm, idst_hbm, out_hbm,
          idx_all, rows0, rows1, tb0, tb1,
          gsem0, gsem1, osem0, osem1):
    wid = lax.axis_index("s") * 2 + lax.axis_index("c")
    b0 = wid * BBLK
    c0tile = wid * 4

    pltpu.sync_copy(idst_hbm.at[:, pl.ds(b0, BBLK)], idx_all)

    iota = lax.iota(jnp.int32, 16)
    bvecs = [[iota + (cl * 128 + bl0 * 16) for bl0 in range(8)]
             for cl in range(2)]

    rows = (rows0, rows1)
    tbs = (tb0, tb1)
    gsem = (gsem0, gsem1)
    osem = (osem0, osem1)

    def idx_ref(chunk_i):
        s = chunk_i // 2
        h = chunk_i % 2
        return idx_all.at[s, pl.ds(h * HALF, HALF)]

    def start_gather(chunk_i, p):
        pltpu.async_copy(table_hbm.at[idx_ref(chunk_i)], rows[p], gsem[p])

    def wait_gather(chunk_i, p):
        pltpu.make_async_copy(table_hbm.at[idx_ref(chunk_i)], rows[p],
                              gsem[p]).wait()

    def out_ref(chunk_i):
        s = chunk_i // 2
        h = chunk_i % 2
        return out_hbm.at[s, :, pl.ds(c0tile + h * 2, 2)]

    def start_write(chunk_i, p):
        pltpu.async_copy(tbs[p], out_ref(chunk_i), osem[p])

    def wait_write(chunk_i, p):
        pltpu.make_async_copy(tbs[p], out_ref(chunk_i), osem[p]).wait()

    def step(gc, p, first):
        wait_gather(gc, p)
        if not first:
            wait_write(gc - 2, p)
        _transpose_chunk(rows[p], tbs[p], bvecs)
        nxt = jnp.minimum(gc + 2, N_CHUNKS - 1)
        start_gather(nxt, p)
        start_write(gc, p)

    start_gather(0, 0)
    start_gather(1, 1)
    step(0, 0, True)
    step(1, 1, True)

    def pair(k, carry):
        g = 2 + 2 * k
        for b in range(2):
            step(g + b, b, False)
        return carry

    lax.fori_loop(0, (N_CHUNKS - 2) // 2, pair, 0)

    wait_gather(N_CHUNKS - 1, 0)
    wait_gather(N_CHUNKS - 1, 1)
    wait_write(N_CHUNKS - 2, 0)
    wait_write(N_CHUNKS - 1, 1)


def kernel(ids, table):
    ids_t = ids.T
    mesh = plsc.VectorSubcoreMesh(core_axis_name="c", subcore_axis_name="s")
    run = pl.kernel(
        _body,
        out_type=jax.ShapeDtypeStruct((SEQ, 8, 128, 1024), jnp.float32),
        mesh=mesh,
        scratch_types=[
            pltpu.VMEM((SEQ, BBLK), jnp.int32),
            pltpu.VMEM((HALF, DIM), jnp.float32),
            pltpu.VMEM((HALF, DIM), jnp.float32),
            pltpu.VMEM((8, 2, 1024), jnp.float32),
            pltpu.VMEM((8, 2, 1024), jnp.float32),
            pltpu.SemaphoreType.DMA,
            pltpu.SemaphoreType.DMA,
            pltpu.SemaphoreType.DMA,
            pltpu.SemaphoreType.DMA,
        ],
        compiler_params=pltpu.CompilerParams(use_tc_tiling_on_sc=False,
                                             needs_layout_passes=False),
    )
    out5 = run(table, ids_t)
    out_t = (out5.reshape(SEQ, 8, 128, 8, 128)
             .transpose(0, 1, 3, 2, 4)
             .reshape(SEQ, DIM, NB))
    return jnp.transpose(out_t, (2, 0, 1))
